# baseline (speedup 1.0000x reference)
import jax
import jax.numpy as jnp
from jax import lax
from jax.experimental import pallas as pl
from jax.experimental.pallas import tpu as pltpu


def _body(ts_ref, x_ref, peB_ref, peBs_ref, peA_ref, peAs_ref, emb_ref,
          out_ref, pe_s):
    S = x_ref.shape[1]
    D = x_ref.shape[2]
    C = D // 4
    j = pl.program_id(1)

    @pl.when(j == 0)
    def _():
        F = peB_ref[...]
        Fs = peBs_ref[...]
        even = (lax.broadcasted_iota(jnp.int32, (256, D), 1) % 2) == 0
        for k in range(S // 256):
            E = peA_ref[k, :][None, :]
            Es = peAs_ref[k, :][None, :]
            blk = (jnp.where(even, Es * F, E * F)
                   + jnp.where(even, E * Fs, -(Es * Fs)))
            pe_s[k * 256:(k + 1) * 256, :] = blk

    ts = ts_ref[0]
    xb = x_ref[0]
    peb = pe_s[...]
    for c in range(4):
        idx = ts[c, :]
        oh = (idx[:, None] == lax.broadcasted_iota(jnp.int32, (S, 32), 1))
        chunk = jnp.dot(oh.astype(jnp.float32),
                        emb_ref[:, c * C:(c + 1) * C],
                        preferred_element_type=jnp.float32)
        out_ref[0, :, c * C:(c + 1) * C] = (
            xb[:, c * C:(c + 1) * C] + peb[:, c * C:(c + 1) * C] + chunk)


def kernel(x, timestamps, pe, hour_emb, day_emb, month_emb, season_emb):
    B, L, D = x.shape
    S = 2048
    nsb = L // S

    pe2 = pe[0]
    tsT = timestamps.transpose(0, 2, 1)

    def swap_pairs(a):
        return a.reshape(a.shape[0], D // 2, 2)[:, :, ::-1].reshape(
            a.shape[0], D)

    peBs = swap_pairs(pe2[:256])
    peA = pe2[0:S:256]
    peAs = swap_pairs(peA)

    def pad32(e):
        return jnp.pad(e, ((0, 32 - e.shape[0]), (0, 0)))

    emb = jnp.concatenate(
        [pad32(hour_emb), pad32(day_emb), pad32(month_emb), pad32(season_emb)],
        axis=1)

    KA = S // 256
    return pl.pallas_call(
        _body,
        grid=(nsb, B),
        in_specs=[
            pl.BlockSpec((1, 4, S), lambda i, j: (j, 0, i)),
            pl.BlockSpec((1, S, D), lambda i, j: (j, i, 0)),
            pl.BlockSpec((256, D), lambda i, j: (0, 0)),
            pl.BlockSpec((256, D), lambda i, j: (0, 0)),
            pl.BlockSpec((KA, D), lambda i, j: (0, 0)),
            pl.BlockSpec((KA, D), lambda i, j: (0, 0)),
            pl.BlockSpec((32, D), lambda i, j: (0, 0)),
        ],
        out_specs=pl.BlockSpec((1, S, D), lambda i, j: (j, i, 0)),
        out_shape=jax.ShapeDtypeStruct((B, L, D), x.dtype),
        scratch_shapes=[pltpu.VMEM((S, D), jnp.float32)],
    )(tsT, x, pe2, peBs, peA, peAs, emb)


# speedup vs baseline: 1.0257x; 1.0257x over previous
import jax
import jax.numpy as jnp
from jax import lax
from jax.experimental import pallas as pl
from jax.experimental.pallas import tpu as pltpu


def _body(ts_ref, x_ref, peB_ref, peBs_ref, peA_ref, peAs_ref, emb_ref,
          out_ref, pe_s):
    S = x_ref.shape[1]
    D = x_ref.shape[2]
    C = D // 4
    j = pl.program_id(1)

    @pl.when(j == 0)
    def _():
        F = peB_ref[...]
        Fs = peBs_ref[...]
        even = (lax.broadcasted_iota(jnp.int32, (256, D), 1) % 2) == 0
        for k in range(S // 256):
            E = peA_ref[k, :][None, :]
            Es = peAs_ref[k, :][None, :]
            blk = (jnp.where(even, Es * F, E * F)
                   + jnp.where(even, E * Fs, -(Es * Fs)))
            pe_s[k * 256:(k + 1) * 256, :] = blk

    ts = ts_ref[0]
    xb = x_ref[0]
    peb = pe_s[...]
    for c in range(4):
        idx = ts[c, :]
        oh = (idx[:, None] == lax.broadcasted_iota(jnp.int32, (S, 32), 1))
        chunk = jnp.dot(oh.astype(jnp.float32),
                        emb_ref[:, c * C:(c + 1) * C],
                        preferred_element_type=jnp.float32)
        out_ref[0, :, c * C:(c + 1) * C] = (
            xb[:, c * C:(c + 1) * C] + peb[:, c * C:(c + 1) * C] + chunk)


def kernel(x, timestamps, pe, hour_emb, day_emb, month_emb, season_emb):
    B, L, D = x.shape
    S = 2048
    nsb = L // S

    pe2 = pe[0]
    tsT = timestamps.transpose(0, 2, 1)

    def swap_pairs(a):
        a3 = a.reshape(a.shape[0], D // 2, 2)
        return jnp.concatenate([a3[:, :, 1:2], a3[:, :, 0:1]],
                               axis=2).reshape(a.shape[0], D)

    peBs = swap_pairs(pe2[:256])
    peA = pe2[0:S:256]
    peAs = swap_pairs(peA)

    def pad32(e):
        return jnp.pad(e, ((0, 32 - e.shape[0]), (0, 0)))

    emb = jnp.concatenate(
        [pad32(hour_emb), pad32(day_emb), pad32(month_emb), pad32(season_emb)],
        axis=1)

    KA = S // 256
    return pl.pallas_call(
        _body,
        grid=(nsb, B),
        in_specs=[
            pl.BlockSpec((1, 4, S), lambda i, j: (j, 0, i)),
            pl.BlockSpec((1, S, D), lambda i, j: (j, i, 0)),
            pl.BlockSpec((256, D), lambda i, j: (0, 0)),
            pl.BlockSpec((256, D), lambda i, j: (0, 0)),
            pl.BlockSpec((KA, D), lambda i, j: (0, 0)),
            pl.BlockSpec((KA, D), lambda i, j: (0, 0)),
            pl.BlockSpec((32, D), lambda i, j: (0, 0)),
        ],
        out_specs=pl.BlockSpec((1, S, D), lambda i, j: (j, i, 0)),
        out_shape=jax.ShapeDtypeStruct((B, L, D), x.dtype),
        scratch_shapes=[pltpu.VMEM((S, D), jnp.float32)],
    )(tsT, x, pe2, peBs, peA, peAs, emb)
